# baseline (speedup 1.0000x reference)
import functools
import math

import jax
import jax.numpy as jnp
from jax import lax
from jax.experimental import pallas as pl
from jax.experimental.pallas import tpu as pltpu
from jax.experimental.pallas import tpu_sc as plsc

ALPHA = 0.1
THETA = 0.5

NC = 2
NS = 16
NW = NC * NS

CH = 128


def _sc_count(dst_rs, ones, zeros, npad, d, nchunk):
  rp = npad // NS

  mesh = plsc.VectorSubcoreMesh(
      core_axis_name="c", subcore_axis_name="s", num_cores=NC, num_subcores=NS)

  @functools.partial(
      pl.kernel,
      mesh=mesh,
      out_type=jax.ShapeDtypeStruct((NC, npad, d), jnp.float32),
      scratch_types=[
          pltpu.VMEM((nchunk, CH), jnp.int32),
          pltpu.VMEM((CH, d), jnp.float32),
          pltpu.VMEM_SHARED((npad, d), jnp.float32),
          pltpu.SemaphoreType.DMA,
      ],
  )
  def k(dst_hbm, ones_hbm, zeros_hbm, out_hbm, dst_v, ones_v, cnt_sh, sem):
    cid = lax.axis_index("c")
    sid = lax.axis_index("s")
    wid = sid * NC + cid
    pltpu.sync_copy(zeros_hbm, cnt_sh.at[pl.ds(sid * rp, rp)])
    pltpu.sync_copy(dst_hbm.at[wid], dst_v)
    pltpu.sync_copy(ones_hbm, ones_v)
    plsc.subcore_barrier()

    def chunk(j, carry):
      pltpu.sync_copy(ones_v, cnt_sh.at[dst_v.at[j]], add=True)
      return carry

    lax.fori_loop(0, nchunk, chunk, 0)
    plsc.subcore_barrier()
    pltpu.sync_copy(cnt_sh.at[pl.ds(sid * rp, rp)],
                    out_hbm.at[cid, pl.ds(sid * rp, rp)])

  return k(dst_rs, ones, zeros)


def _sc_spmm(hp, src_rs, dst_rs, zeros, npad, d, nchunk):
  rp = npad // NS

  mesh = plsc.VectorSubcoreMesh(
      core_axis_name="c", subcore_axis_name="s", num_cores=NC, num_subcores=NS)

  @functools.partial(
      pl.kernel,
      mesh=mesh,
      out_type=jax.ShapeDtypeStruct((NC, npad, d), jnp.float32),
      scratch_types=[
          pltpu.VMEM((CH,), jnp.int32),
          pltpu.VMEM((CH,), jnp.int32),
          pltpu.VMEM((CH,), jnp.int32),
          pltpu.VMEM((CH,), jnp.int32),
          pltpu.VMEM((CH, d), jnp.float32),
          pltpu.VMEM((CH, d), jnp.float32),
          pltpu.VMEM_SHARED((npad, d), jnp.float32),
          pltpu.SemaphoreType.DMA,
          pltpu.SemaphoreType.DMA,
          pltpu.SemaphoreType.DMA,
          pltpu.SemaphoreType.DMA,
      ],
  )
  def k(hp_hbm, src_hbm, dst_hbm, zeros_hbm, out_hbm,
        si0, si1, di0, di1, rows0_v, rows1_v, agg_sh, is0, is1, gs0, gs1):
    cid = lax.axis_index("c")
    sid = lax.axis_index("s")
    wid = sid * NC + cid
    si = (si0, si1)
    di = (di0, di1)
    rows = (rows0_v, rows1_v)
    isem = (is0, is1)
    gsem = (gs0, gs1)
    pltpu.sync_copy(zeros_hbm, agg_sh.at[pl.ds(sid * rp, rp)])
    plsc.subcore_barrier()

    pltpu.async_copy(src_hbm.at[wid, 0], si0, is0)
    pltpu.async_copy(dst_hbm.at[wid, 0], di0, is0)

    def step(j, carry):
      b = lax.rem(j, 2)

      def stage(bb):
        sib, dib, rob = si[bb], di[bb], rows[bb]
        sio, dio, roo = si[1 - bb], di[1 - bb], rows[1 - bb]
        ise, gse = isem[bb], gsem[bb]

        pltpu.make_async_copy(src_hbm.at[wid, j], sib, ise).wait()
        pltpu.make_async_copy(dst_hbm.at[wid, j], dib, ise).wait()
        pltpu.async_copy(hp_hbm.at[sib], rob, gse)

        @pl.when(j > 0)
        def _():
          pltpu.make_async_copy(hp_hbm.at[sio], roo, gsem[1 - bb]).wait()
          pltpu.sync_copy(roo, agg_sh.at[dio], add=True)

        @pl.when(j + 1 < nchunk)
        def _():
          pltpu.async_copy(src_hbm.at[wid, j + 1], sio, isem[1 - bb])
          pltpu.async_copy(dst_hbm.at[wid, j + 1], dio, isem[1 - bb])

      lax.cond(b == 0, lambda: stage(0), lambda: stage(1))
      return carry

    lax.fori_loop(0, nchunk, step, 0)
    last = (nchunk - 1) % 2
    pltpu.make_async_copy(hp_hbm.at[si[last]], rows[last], gsem[last]).wait()
    pltpu.sync_copy(rows[last], agg_sh.at[di[last]], add=True)
    plsc.subcore_barrier()
    pltpu.sync_copy(agg_sh.at[pl.ds(sid * rp, rp)],
                    out_hbm.at[cid, pl.ds(sid * rp, rp)])

  return k(hp, src_rs, dst_rs, zeros)


def _tc_prep(x, w1, b1, cnt, n, d, h, bm):
  def body(x_ref, w_ref, b_ref, cnt_ref, h0_ref, hp_ref, dinv_ref):
    deg = cnt_ref[0, :, 0:1] + cnt_ref[1, :, 0:1] + 1.0
    di = lax.rsqrt(deg)
    hv = jnp.maximum(
        jnp.dot(x_ref[...], w_ref[...], preferred_element_type=jnp.float32)
        + b_ref[...], 0.0)
    h0_ref[...] = hv
    hp_ref[...] = hv * di
    dinv_ref[...] = di

  grid = (n // bm,)
  return pl.pallas_call(
      body,
      grid=grid,
      in_specs=[
          pl.BlockSpec((bm, d), lambda i: (i, 0)),
          pl.BlockSpec((d, h), lambda i: (0, 0)),
          pl.BlockSpec((1, h), lambda i: (0, 0)),
          pl.BlockSpec((NC, bm, 128), lambda i: (0, i, 0)),
      ],
      out_specs=[
          pl.BlockSpec((bm, h), lambda i: (i, 0)),
          pl.BlockSpec((bm, h), lambda i: (i, 0)),
          pl.BlockSpec((bm, 1), lambda i: (i, 0)),
      ],
      out_shape=[
          jax.ShapeDtypeStruct((n, h), jnp.float32),
          jax.ShapeDtypeStruct((n, h), jnp.float32),
          jax.ShapeDtypeStruct((n, 1), jnp.float32),
      ],
  )(x, w1, b1, cnt)


def _tc_layer(s2, hp, x0, dinv, w, beta, n, h, bm):
  def body(s_ref, hp_ref, x0_ref, di_ref, w_ref, h_ref, hpn_ref):
    di = di_ref[...]
    agg = di * (s_ref[0] + s_ref[1] + hp_ref[...])
    hh = (1.0 - ALPHA) * agg + ALPHA * x0_ref[...]
    out = (1.0 - beta) * hh + beta * jnp.dot(
        hh, w_ref[...], preferred_element_type=jnp.float32)
    hnew = jnp.maximum(out, 0.0)
    h_ref[...] = hnew
    hpn_ref[...] = hnew * di

  grid = (n // bm,)
  return pl.pallas_call(
      body,
      grid=grid,
      in_specs=[
          pl.BlockSpec((NC, bm, h), lambda i: (0, i, 0)),
          pl.BlockSpec((bm, h), lambda i: (i, 0)),
          pl.BlockSpec((bm, h), lambda i: (i, 0)),
          pl.BlockSpec((bm, 1), lambda i: (i, 0)),
          pl.BlockSpec((h, h), lambda i: (0, 0)),
      ],
      out_specs=[
          pl.BlockSpec((bm, h), lambda i: (i, 0)),
          pl.BlockSpec((bm, h), lambda i: (i, 0)),
      ],
      out_shape=[
          jax.ShapeDtypeStruct((n, h), jnp.float32),
          jax.ShapeDtypeStruct((n, h), jnp.float32),
      ],
  )(s2, hp, x0, dinv, w)


def _tc_pool(hfin, batch2, w2, b2, n, h, c, g):
  def body(h_ref, b_ref, w_ref, bias_ref, out_ref):
    seg = lax.broadcasted_iota(jnp.int32, (g, n), 0)
    oh = (seg == b_ref[...]).astype(jnp.float32)
    sums = jnp.dot(oh, h_ref[...], preferred_element_type=jnp.float32)
    counts = jnp.sum(oh, axis=1, keepdims=True)
    pooled = sums / jnp.maximum(counts, 1.0)
    logits = jnp.dot(pooled, w_ref[...],
                     preferred_element_type=jnp.float32) + bias_ref[...]
    m = jnp.max(logits, axis=-1, keepdims=True)
    z = logits - m
    lse = jnp.log(jnp.sum(jnp.exp(z), axis=-1, keepdims=True))
    out_ref[...] = z - lse

  return pl.pallas_call(
      body,
      out_shape=jax.ShapeDtypeStruct((g, c), jnp.float32),
  )(hfin, batch2, w2, b2)


@jax.jit
def kernel(x, lin1_w, lin1_b, conv_w, lin2_w, lin2_b, edge_index, batch):
  n, d = x.shape
  h = lin1_w.shape[1]
  nlayers = conv_w.shape[0]
  c = lin2_w.shape[1]
  g = 64
  e = edge_index.shape[1]
  bm = n // 10
  npad = 10240

  nchunk = -(-e // (NW * CH))
  epad = NW * nchunk * CH - e
  src_flat = jnp.concatenate(
      [edge_index[0].astype(jnp.int32),
       jnp.zeros((epad,), jnp.int32)])
  dst_flat = jnp.concatenate(
      [edge_index[1].astype(jnp.int32),
       jnp.full((epad,), npad - 1, jnp.int32)])
  src_rs = src_flat.reshape(NW, nchunk, CH)
  dst_rs = dst_flat.reshape(NW, nchunk, CH)
  onesd = jnp.ones((CH, h), jnp.float32)
  zerosd = jnp.zeros((npad // NS, h), jnp.float32)

  cnt = _sc_count(dst_rs, onesd, zerosd, npad, h, nchunk)
  h0, hp, dinv = _tc_prep(x, lin1_w, lin1_b.reshape(1, h), cnt, n, d, h, bm)

  hcur, hpcur = h0, hp
  for l in range(nlayers):
    beta = float(math.log(THETA / (l + 1) + 1.0))
    s2 = _sc_spmm(hpcur, src_rs, dst_rs, zerosd, npad, h, nchunk)
    hcur, hpcur = _tc_layer(s2, hpcur, h0, dinv, conv_w[l], beta, n, h, bm)

  return _tc_pool(hcur, batch.reshape(1, n).astype(jnp.int32), lin2_w,
                  lin2_b.reshape(1, c), n, h, c, g)


# speedup vs baseline: 1.1853x; 1.1853x over previous
import functools
import math

import jax
import jax.numpy as jnp
from jax import lax
from jax.experimental import pallas as pl
from jax.experimental.pallas import tpu as pltpu
from jax.experimental.pallas import tpu_sc as plsc

ALPHA = 0.1
THETA = 0.5

NC = 2
NS = 16
NW = NC * NS

CH = 96


def _sc_count(dst_rs, ones, zeros, npad, d, nchunk):
  rp = npad // NS

  mesh = plsc.VectorSubcoreMesh(
      core_axis_name="c", subcore_axis_name="s", num_cores=NC, num_subcores=NS)

  @functools.partial(
      pl.kernel,
      mesh=mesh,
      out_type=jax.ShapeDtypeStruct((NC, npad, d), jnp.float32),
      scratch_types=[
          pltpu.VMEM((nchunk, CH), jnp.int32),
          pltpu.VMEM((CH, d), jnp.float32),
          pltpu.VMEM_SHARED((npad, d), jnp.float32),
          pltpu.SemaphoreType.DMA,
      ],
  )
  def k(dst_hbm, ones_hbm, zeros_hbm, out_hbm, dst_v, ones_v, cnt_sh, sem):
    cid = lax.axis_index("c")
    sid = lax.axis_index("s")
    wid = sid * NC + cid
    pltpu.sync_copy(zeros_hbm, cnt_sh.at[pl.ds(sid * rp, rp)])
    pltpu.sync_copy(dst_hbm.at[wid], dst_v)
    pltpu.sync_copy(ones_hbm, ones_v)
    plsc.subcore_barrier()

    def chunk(j, carry):
      pltpu.sync_copy(ones_v, cnt_sh.at[dst_v.at[j]], add=True)
      return carry

    lax.fori_loop(0, nchunk, chunk, 0)
    plsc.subcore_barrier()
    pltpu.sync_copy(cnt_sh.at[pl.ds(sid * rp, rp)],
                    out_hbm.at[cid, pl.ds(sid * rp, rp)])

  return k(dst_rs, ones, zeros)


def _sc_spmm(hp, src_rs, dst_rs, zeros, npad, d, nchunk):
  rp = npad // NS

  mesh = plsc.VectorSubcoreMesh(
      core_axis_name="c", subcore_axis_name="s", num_cores=NC, num_subcores=NS)

  @functools.partial(
      pl.kernel,
      mesh=mesh,
      out_type=jax.ShapeDtypeStruct((NC, npad, d), jnp.float32),
      scratch_types=[
          pltpu.VMEM((CH,), jnp.int32),
          pltpu.VMEM((CH,), jnp.int32),
          pltpu.VMEM((CH,), jnp.int32),
          pltpu.VMEM((CH,), jnp.int32),
          pltpu.VMEM((CH, d), jnp.float32),
          pltpu.VMEM((CH, d), jnp.float32),
          pltpu.VMEM_SHARED((npad, d), jnp.float32),
          pltpu.SemaphoreType.DMA,
          pltpu.SemaphoreType.DMA,
          pltpu.SemaphoreType.DMA,
          pltpu.SemaphoreType.DMA,
      ],
  )
  def k(hp_hbm, src_hbm, dst_hbm, zeros_hbm, out_hbm,
        si0, si1, di0, di1, rows0_v, rows1_v, agg_sh, is0, is1, gs0, gs1):
    cid = lax.axis_index("c")
    sid = lax.axis_index("s")
    wid = sid * NC + cid
    si = (si0, si1)
    di = (di0, di1)
    rows = (rows0_v, rows1_v)
    isem = (is0, is1)
    gsem = (gs0, gs1)
    pltpu.sync_copy(zeros_hbm, agg_sh.at[pl.ds(sid * rp, rp)])
    plsc.subcore_barrier()

    pltpu.async_copy(src_hbm.at[wid, 0], si0, is0)
    pltpu.async_copy(dst_hbm.at[wid, 0], di0, is0)

    def step(j, carry):
      b = lax.rem(j, 2)

      def stage(bb):
        sib, dib, rob = si[bb], di[bb], rows[bb]
        sio, dio, roo = si[1 - bb], di[1 - bb], rows[1 - bb]
        ise, gse = isem[bb], gsem[bb]

        pltpu.make_async_copy(src_hbm.at[wid, j], sib, ise).wait()
        pltpu.make_async_copy(dst_hbm.at[wid, j], dib, ise).wait()
        pltpu.async_copy(hp_hbm.at[sib], rob, gse)

        @pl.when(j > 0)
        def _():
          pltpu.make_async_copy(hp_hbm.at[sio], roo, gsem[1 - bb]).wait()
          pltpu.sync_copy(roo, agg_sh.at[dio], add=True)

        @pl.when(j + 1 < nchunk)
        def _():
          pltpu.async_copy(src_hbm.at[wid, j + 1], sio, isem[1 - bb])
          pltpu.async_copy(dst_hbm.at[wid, j + 1], dio, isem[1 - bb])

      lax.cond(b == 0, lambda: stage(0), lambda: stage(1))
      return carry

    lax.fori_loop(0, nchunk, step, 0)
    last = (nchunk - 1) % 2
    pltpu.make_async_copy(hp_hbm.at[si[last]], rows[last], gsem[last]).wait()
    pltpu.sync_copy(rows[last], agg_sh.at[di[last]], add=True)
    plsc.subcore_barrier()
    pltpu.sync_copy(agg_sh.at[pl.ds(sid * rp, rp)],
                    out_hbm.at[cid, pl.ds(sid * rp, rp)])

  return k(hp, src_rs, dst_rs, zeros)


def _tc_prep(x, w1, b1, cnt, n, d, h, bm):
  def body(x_ref, w_ref, b_ref, cnt_ref, h0_ref, hp_ref, dinv_ref):
    deg = cnt_ref[0, :, 0:1] + cnt_ref[1, :, 0:1] + 1.0
    di = lax.rsqrt(deg)
    hv = jnp.maximum(
        jnp.dot(x_ref[...], w_ref[...], preferred_element_type=jnp.float32)
        + b_ref[...], 0.0)
    h0_ref[...] = hv
    hp_ref[...] = hv * di
    dinv_ref[...] = di

  grid = (n // bm,)
  return pl.pallas_call(
      body,
      grid=grid,
      in_specs=[
          pl.BlockSpec((bm, d), lambda i: (i, 0)),
          pl.BlockSpec((d, h), lambda i: (0, 0)),
          pl.BlockSpec((1, h), lambda i: (0, 0)),
          pl.BlockSpec((NC, bm, 128), lambda i: (0, i, 0)),
      ],
      out_specs=[
          pl.BlockSpec((bm, h), lambda i: (i, 0)),
          pl.BlockSpec((bm, h), lambda i: (i, 0)),
          pl.BlockSpec((bm, 1), lambda i: (i, 0)),
      ],
      out_shape=[
          jax.ShapeDtypeStruct((n, h), jnp.float32),
          jax.ShapeDtypeStruct((n, h), jnp.float32),
          jax.ShapeDtypeStruct((n, 1), jnp.float32),
      ],
  )(x, w1, b1, cnt)


def _tc_layer(s2, hp, x0, dinv, w, beta, n, h, bm):
  def body(s_ref, hp_ref, x0_ref, di_ref, w_ref, h_ref, hpn_ref):
    di = di_ref[...]
    agg = di * (s_ref[0] + s_ref[1] + hp_ref[...])
    hh = (1.0 - ALPHA) * agg + ALPHA * x0_ref[...]
    out = (1.0 - beta) * hh + beta * jnp.dot(
        hh, w_ref[...], preferred_element_type=jnp.float32)
    hnew = jnp.maximum(out, 0.0)
    h_ref[...] = hnew
    hpn_ref[...] = hnew * di

  grid = (n // bm,)
  return pl.pallas_call(
      body,
      grid=grid,
      in_specs=[
          pl.BlockSpec((NC, bm, h), lambda i: (0, i, 0)),
          pl.BlockSpec((bm, h), lambda i: (i, 0)),
          pl.BlockSpec((bm, h), lambda i: (i, 0)),
          pl.BlockSpec((bm, 1), lambda i: (i, 0)),
          pl.BlockSpec((h, h), lambda i: (0, 0)),
      ],
      out_specs=[
          pl.BlockSpec((bm, h), lambda i: (i, 0)),
          pl.BlockSpec((bm, h), lambda i: (i, 0)),
      ],
      out_shape=[
          jax.ShapeDtypeStruct((n, h), jnp.float32),
          jax.ShapeDtypeStruct((n, h), jnp.float32),
      ],
  )(s2, hp, x0, dinv, w)


def _tc_pool(hfin, batch2, w2, b2, n, h, c, g):
  def body(h_ref, b_ref, w_ref, bias_ref, out_ref):
    seg = lax.broadcasted_iota(jnp.int32, (g, n), 0)
    oh = (seg == b_ref[...]).astype(jnp.float32)
    sums = jnp.dot(oh, h_ref[...], preferred_element_type=jnp.float32)
    counts = jnp.sum(oh, axis=1, keepdims=True)
    pooled = sums / jnp.maximum(counts, 1.0)
    logits = jnp.dot(pooled, w_ref[...],
                     preferred_element_type=jnp.float32) + bias_ref[...]
    m = jnp.max(logits, axis=-1, keepdims=True)
    z = logits - m
    lse = jnp.log(jnp.sum(jnp.exp(z), axis=-1, keepdims=True))
    out_ref[...] = z - lse

  return pl.pallas_call(
      body,
      out_shape=jax.ShapeDtypeStruct((g, c), jnp.float32),
  )(hfin, batch2, w2, b2)


@jax.jit
def kernel(x, lin1_w, lin1_b, conv_w, lin2_w, lin2_b, edge_index, batch):
  n, d = x.shape
  h = lin1_w.shape[1]
  nlayers = conv_w.shape[0]
  c = lin2_w.shape[1]
  g = 64
  e = edge_index.shape[1]
  bm = n // 10
  npad = 10240

  nchunk = -(-e // (NW * CH))
  epad = NW * nchunk * CH - e
  src_flat = jnp.concatenate(
      [edge_index[0].astype(jnp.int32),
       jnp.zeros((epad,), jnp.int32)])
  dst_flat = jnp.concatenate(
      [edge_index[1].astype(jnp.int32),
       jnp.full((epad,), npad - 1, jnp.int32)])
  src_rs = src_flat.reshape(NW, nchunk, CH)
  dst_rs = dst_flat.reshape(NW, nchunk, CH)
  onesd = jnp.ones((CH, h), jnp.float32)
  zerosd = jnp.zeros((npad // NS, h), jnp.float32)

  cnt = _sc_count(dst_rs, onesd, zerosd, npad, h, nchunk)
  h0, hp, dinv = _tc_prep(x, lin1_w, lin1_b.reshape(1, h), cnt, n, d, h, bm)

  hcur, hpcur = h0, hp
  for l in range(nlayers):
    beta = float(math.log(THETA / (l + 1) + 1.0))
    s2 = _sc_spmm(hpcur, src_rs, dst_rs, zerosd, npad, h, nchunk)
    hcur, hpcur = _tc_layer(s2, hpcur, h0, dinv, conv_w[l], beta, n, h, bm)

  return _tc_pool(hcur, batch.reshape(1, n).astype(jnp.int32), lin2_w,
                  lin2_b.reshape(1, c), n, h, c, g)


# speedup vs baseline: 1.2630x; 1.0655x over previous
import functools
import math

import jax
import jax.numpy as jnp
from jax import lax
from jax.experimental import pallas as pl
from jax.experimental.pallas import tpu as pltpu
from jax.experimental.pallas import tpu_sc as plsc

ALPHA = 0.1
THETA = 0.5

NC = 2
NS = 16
NW = NC * NS

CH = 64


def _sc_count(dst_rs, ones, zeros, npad, d, nchunk):
  rp = npad // NS

  mesh = plsc.VectorSubcoreMesh(
      core_axis_name="c", subcore_axis_name="s", num_cores=NC, num_subcores=NS)

  @functools.partial(
      pl.kernel,
      mesh=mesh,
      out_type=jax.ShapeDtypeStruct((NC, npad, d), jnp.float32),
      scratch_types=[
          pltpu.VMEM((nchunk, CH), jnp.int32),
          pltpu.VMEM((CH, d), jnp.float32),
          pltpu.VMEM_SHARED((npad, d), jnp.float32),
          pltpu.SemaphoreType.DMA,
      ],
  )
  def k(dst_hbm, ones_hbm, zeros_hbm, out_hbm, dst_v, ones_v, cnt_sh, sem):
    cid = lax.axis_index("c")
    sid = lax.axis_index("s")
    wid = sid * NC + cid
    pltpu.sync_copy(zeros_hbm, cnt_sh.at[pl.ds(sid * rp, rp)])
    pltpu.sync_copy(dst_hbm.at[wid], dst_v)
    pltpu.sync_copy(ones_hbm, ones_v)
    plsc.subcore_barrier()

    def chunk(j, carry):
      pltpu.sync_copy(ones_v, cnt_sh.at[dst_v.at[j]], add=True)
      return carry

    lax.fori_loop(0, nchunk, chunk, 0)
    plsc.subcore_barrier()
    pltpu.sync_copy(cnt_sh.at[pl.ds(sid * rp, rp)],
                    out_hbm.at[cid, pl.ds(sid * rp, rp)])

  return k(dst_rs, ones, zeros)


def _sc_spmm(hp, src_rs, dst_rs, zeros, npad, d, nchunk):
  rp = npad // NS

  mesh = plsc.VectorSubcoreMesh(
      core_axis_name="c", subcore_axis_name="s", num_cores=NC, num_subcores=NS)

  @functools.partial(
      pl.kernel,
      mesh=mesh,
      out_type=jax.ShapeDtypeStruct((NC, npad, d), jnp.float32),
      scratch_types=[
          pltpu.VMEM((CH,), jnp.int32),
          pltpu.VMEM((CH,), jnp.int32),
          pltpu.VMEM((CH,), jnp.int32),
          pltpu.VMEM((CH,), jnp.int32),
          pltpu.VMEM((CH, d), jnp.float32),
          pltpu.VMEM((CH, d), jnp.float32),
          pltpu.VMEM_SHARED((npad, d), jnp.float32),
          pltpu.SemaphoreType.DMA,
          pltpu.SemaphoreType.DMA,
          pltpu.SemaphoreType.DMA,
          pltpu.SemaphoreType.DMA,
      ],
  )
  def k(hp_hbm, src_hbm, dst_hbm, zeros_hbm, out_hbm,
        si0, si1, di0, di1, rows0_v, rows1_v, agg_sh, is0, is1, gs0, gs1):
    cid = lax.axis_index("c")
    sid = lax.axis_index("s")
    wid = sid * NC + cid
    si = (si0, si1)
    di = (di0, di1)
    rows = (rows0_v, rows1_v)
    isem = (is0, is1)
    gsem = (gs0, gs1)
    pltpu.sync_copy(zeros_hbm, agg_sh.at[pl.ds(sid * rp, rp)])
    plsc.subcore_barrier()

    pltpu.async_copy(src_hbm.at[wid, 0], si0, is0)
    pltpu.async_copy(dst_hbm.at[wid, 0], di0, is0)

    def step(j, carry):
      b = lax.rem(j, 2)

      def stage(bb):
        sib, dib, rob = si[bb], di[bb], rows[bb]
        sio, dio, roo = si[1 - bb], di[1 - bb], rows[1 - bb]
        ise, gse = isem[bb], gsem[bb]

        pltpu.make_async_copy(src_hbm.at[wid, j], sib, ise).wait()
        pltpu.make_async_copy(dst_hbm.at[wid, j], dib, ise).wait()
        pltpu.async_copy(hp_hbm.at[sib], rob, gse)

        @pl.when(j > 0)
        def _():
          pltpu.make_async_copy(hp_hbm.at[sio], roo, gsem[1 - bb]).wait()
          pltpu.sync_copy(roo, agg_sh.at[dio], add=True)

        @pl.when(j + 1 < nchunk)
        def _():
          pltpu.async_copy(src_hbm.at[wid, j + 1], sio, isem[1 - bb])
          pltpu.async_copy(dst_hbm.at[wid, j + 1], dio, isem[1 - bb])

      lax.cond(b == 0, lambda: stage(0), lambda: stage(1))
      return carry

    lax.fori_loop(0, nchunk, step, 0)
    last = (nchunk - 1) % 2
    pltpu.make_async_copy(hp_hbm.at[si[last]], rows[last], gsem[last]).wait()
    pltpu.sync_copy(rows[last], agg_sh.at[di[last]], add=True)
    plsc.subcore_barrier()
    pltpu.sync_copy(agg_sh.at[pl.ds(sid * rp, rp)],
                    out_hbm.at[cid, pl.ds(sid * rp, rp)])

  return k(hp, src_rs, dst_rs, zeros)


def _tc_prep(x, w1, b1, cnt, n, d, h, bm):
  def body(x_ref, w_ref, b_ref, cnt_ref, h0_ref, hp_ref, dinv_ref):
    deg = cnt_ref[0, :, 0:1] + cnt_ref[1, :, 0:1] + 1.0
    di = lax.rsqrt(deg)
    hv = jnp.maximum(
        jnp.dot(x_ref[...], w_ref[...], preferred_element_type=jnp.float32)
        + b_ref[...], 0.0)
    h0_ref[...] = hv
    hp_ref[...] = hv * di
    dinv_ref[...] = di

  grid = (n // bm,)
  return pl.pallas_call(
      body,
      grid=grid,
      in_specs=[
          pl.BlockSpec((bm, d), lambda i: (i, 0)),
          pl.BlockSpec((d, h), lambda i: (0, 0)),
          pl.BlockSpec((1, h), lambda i: (0, 0)),
          pl.BlockSpec((NC, bm, 128), lambda i: (0, i, 0)),
      ],
      out_specs=[
          pl.BlockSpec((bm, h), lambda i: (i, 0)),
          pl.BlockSpec((bm, h), lambda i: (i, 0)),
          pl.BlockSpec((bm, 1), lambda i: (i, 0)),
      ],
      out_shape=[
          jax.ShapeDtypeStruct((n, h), jnp.float32),
          jax.ShapeDtypeStruct((n, h), jnp.float32),
          jax.ShapeDtypeStruct((n, 1), jnp.float32),
      ],
  )(x, w1, b1, cnt)


def _tc_layer(s2, hp, x0, dinv, w, beta, n, h, bm):
  def body(s_ref, hp_ref, x0_ref, di_ref, w_ref, h_ref, hpn_ref):
    di = di_ref[...]
    agg = di * (s_ref[0] + s_ref[1] + hp_ref[...])
    hh = (1.0 - ALPHA) * agg + ALPHA * x0_ref[...]
    out = (1.0 - beta) * hh + beta * jnp.dot(
        hh, w_ref[...], preferred_element_type=jnp.float32)
    hnew = jnp.maximum(out, 0.0)
    h_ref[...] = hnew
    hpn_ref[...] = hnew * di

  grid = (n // bm,)
  return pl.pallas_call(
      body,
      grid=grid,
      in_specs=[
          pl.BlockSpec((NC, bm, h), lambda i: (0, i, 0)),
          pl.BlockSpec((bm, h), lambda i: (i, 0)),
          pl.BlockSpec((bm, h), lambda i: (i, 0)),
          pl.BlockSpec((bm, 1), lambda i: (i, 0)),
          pl.BlockSpec((h, h), lambda i: (0, 0)),
      ],
      out_specs=[
          pl.BlockSpec((bm, h), lambda i: (i, 0)),
          pl.BlockSpec((bm, h), lambda i: (i, 0)),
      ],
      out_shape=[
          jax.ShapeDtypeStruct((n, h), jnp.float32),
          jax.ShapeDtypeStruct((n, h), jnp.float32),
      ],
  )(s2, hp, x0, dinv, w)


def _tc_pool(hfin, batch2, w2, b2, n, h, c, g):
  def body(h_ref, b_ref, w_ref, bias_ref, out_ref):
    seg = lax.broadcasted_iota(jnp.int32, (g, n), 0)
    oh = (seg == b_ref[...]).astype(jnp.float32)
    sums = jnp.dot(oh, h_ref[...], preferred_element_type=jnp.float32)
    counts = jnp.sum(oh, axis=1, keepdims=True)
    pooled = sums / jnp.maximum(counts, 1.0)
    logits = jnp.dot(pooled, w_ref[...],
                     preferred_element_type=jnp.float32) + bias_ref[...]
    m = jnp.max(logits, axis=-1, keepdims=True)
    z = logits - m
    lse = jnp.log(jnp.sum(jnp.exp(z), axis=-1, keepdims=True))
    out_ref[...] = z - lse

  return pl.pallas_call(
      body,
      out_shape=jax.ShapeDtypeStruct((g, c), jnp.float32),
  )(hfin, batch2, w2, b2)


@jax.jit
def kernel(x, lin1_w, lin1_b, conv_w, lin2_w, lin2_b, edge_index, batch):
  n, d = x.shape
  h = lin1_w.shape[1]
  nlayers = conv_w.shape[0]
  c = lin2_w.shape[1]
  g = 64
  e = edge_index.shape[1]
  bm = n // 10
  npad = 10240

  nchunk = -(-e // (NW * CH))
  epad = NW * nchunk * CH - e
  src_flat = jnp.concatenate(
      [edge_index[0].astype(jnp.int32),
       jnp.zeros((epad,), jnp.int32)])
  dst_flat = jnp.concatenate(
      [edge_index[1].astype(jnp.int32),
       jnp.full((epad,), npad - 1, jnp.int32)])
  src_rs = src_flat.reshape(NW, nchunk, CH)
  dst_rs = dst_flat.reshape(NW, nchunk, CH)
  onesd = jnp.ones((CH, h), jnp.float32)
  zerosd = jnp.zeros((npad // NS, h), jnp.float32)

  cnt = _sc_count(dst_rs, onesd, zerosd, npad, h, nchunk)
  h0, hp, dinv = _tc_prep(x, lin1_w, lin1_b.reshape(1, h), cnt, n, d, h, bm)

  hcur, hpcur = h0, hp
  for l in range(nlayers):
    beta = float(math.log(THETA / (l + 1) + 1.0))
    s2 = _sc_spmm(hpcur, src_rs, dst_rs, zerosd, npad, h, nchunk)
    hcur, hpcur = _tc_layer(s2, hpcur, h0, dinv, conv_w[l], beta, n, h, bm)

  return _tc_pool(hcur, batch.reshape(1, n).astype(jnp.int32), lin2_w,
                  lin2_b.reshape(1, c), n, h, c, g)


# speedup vs baseline: 1.7590x; 1.3927x over previous
import functools
import math

import jax
import jax.numpy as jnp
from jax import lax
from jax.experimental import pallas as pl
from jax.experimental.pallas import tpu as pltpu
from jax.experimental.pallas import tpu_sc as plsc

ALPHA = 0.1
THETA = 0.5

NC = 2
NS = 16
NW = NC * NS

CH = 80


def _sc_count(dst_rs, ones, zeros, npad, d, nchunk):
  rp = npad // NS

  mesh = plsc.VectorSubcoreMesh(
      core_axis_name="c", subcore_axis_name="s", num_cores=NC, num_subcores=NS)

  @functools.partial(
      pl.kernel,
      mesh=mesh,
      out_type=jax.ShapeDtypeStruct((NC, npad, d), jnp.float32),
      scratch_types=[
          pltpu.VMEM((nchunk, CH), jnp.int32),
          pltpu.VMEM((CH, d), jnp.float32),
          pltpu.VMEM_SHARED((npad, d), jnp.float32),
          pltpu.SemaphoreType.DMA,
      ],
  )
  def k(dst_hbm, ones_hbm, zeros_hbm, out_hbm, dst_v, ones_v, cnt_sh, sem):
    cid = lax.axis_index("c")
    sid = lax.axis_index("s")
    wid = sid * NC + cid
    pltpu.sync_copy(zeros_hbm, cnt_sh.at[pl.ds(sid * rp, rp)])
    pltpu.sync_copy(dst_hbm.at[wid], dst_v)
    pltpu.sync_copy(ones_hbm, ones_v)
    plsc.subcore_barrier()

    def chunk(j, carry):
      pltpu.sync_copy(ones_v, cnt_sh.at[dst_v.at[j]], add=True)
      return carry

    lax.fori_loop(0, nchunk, chunk, 0)
    plsc.subcore_barrier()
    pltpu.sync_copy(cnt_sh.at[pl.ds(sid * rp, rp)],
                    out_hbm.at[cid, pl.ds(sid * rp, rp)])

  return k(dst_rs, ones, zeros)


def _sc_spmm(hp, src_rs, dst_rs, zeros, npad, d, nchunk):
  rp = npad // NS

  mesh = plsc.VectorSubcoreMesh(
      core_axis_name="c", subcore_axis_name="s", num_cores=NC, num_subcores=NS)

  @functools.partial(
      pl.kernel,
      mesh=mesh,
      out_type=jax.ShapeDtypeStruct((NC, npad, d), jnp.float32),
      scratch_types=[
          pltpu.VMEM((CH,), jnp.int32),
          pltpu.VMEM((CH,), jnp.int32),
          pltpu.VMEM((CH,), jnp.int32),
          pltpu.VMEM((CH,), jnp.int32),
          pltpu.VMEM((CH, d), jnp.float32),
          pltpu.VMEM((CH, d), jnp.float32),
          pltpu.VMEM_SHARED((npad, d), jnp.float32),
          pltpu.SemaphoreType.DMA,
          pltpu.SemaphoreType.DMA,
          pltpu.SemaphoreType.DMA,
          pltpu.SemaphoreType.DMA,
      ],
  )
  def k(hp_hbm, src_hbm, dst_hbm, zeros_hbm, out_hbm,
        si0, si1, di0, di1, rows0_v, rows1_v, agg_sh, is0, is1, gs0, gs1):
    cid = lax.axis_index("c")
    sid = lax.axis_index("s")
    wid = sid * NC + cid
    si = (si0, si1)
    di = (di0, di1)
    rows = (rows0_v, rows1_v)
    isem = (is0, is1)
    gsem = (gs0, gs1)
    pltpu.sync_copy(zeros_hbm, agg_sh.at[pl.ds(sid * rp, rp)])
    plsc.subcore_barrier()

    pltpu.async_copy(src_hbm.at[wid, 0], si0, is0)
    pltpu.async_copy(dst_hbm.at[wid, 0], di0, is0)

    def step(j, carry):
      b = lax.rem(j, 2)

      def stage(bb):
        sib, dib, rob = si[bb], di[bb], rows[bb]
        sio, dio, roo = si[1 - bb], di[1 - bb], rows[1 - bb]
        ise, gse = isem[bb], gsem[bb]

        pltpu.make_async_copy(src_hbm.at[wid, j], sib, ise).wait()
        pltpu.make_async_copy(dst_hbm.at[wid, j], dib, ise).wait()
        pltpu.async_copy(hp_hbm.at[sib], rob, gse)

        @pl.when(j > 0)
        def _():
          pltpu.make_async_copy(hp_hbm.at[sio], roo, gsem[1 - bb]).wait()
          pltpu.sync_copy(roo, agg_sh.at[dio], add=True)

        @pl.when(j + 1 < nchunk)
        def _():
          pltpu.async_copy(src_hbm.at[wid, j + 1], sio, isem[1 - bb])
          pltpu.async_copy(dst_hbm.at[wid, j + 1], dio, isem[1 - bb])

      lax.cond(b == 0, lambda: stage(0), lambda: stage(1))
      return carry

    lax.fori_loop(0, nchunk, step, 0)
    last = (nchunk - 1) % 2
    pltpu.make_async_copy(hp_hbm.at[si[last]], rows[last], gsem[last]).wait()
    pltpu.sync_copy(rows[last], agg_sh.at[di[last]], add=True)
    plsc.subcore_barrier()
    pltpu.sync_copy(agg_sh.at[pl.ds(sid * rp, rp)],
                    out_hbm.at[cid, pl.ds(sid * rp, rp)])

  return k(hp, src_rs, dst_rs, zeros)


def _tc_prep(x, w1, b1, cnt, n, d, h, bm):
  def body(x_ref, w_ref, b_ref, cnt_ref, h0_ref, hp_ref, dinv_ref):
    deg = cnt_ref[0, :, 0:1] + cnt_ref[1, :, 0:1] + 1.0
    di = lax.rsqrt(deg)
    hv = jnp.maximum(
        jnp.dot(x_ref[...], w_ref[...], preferred_element_type=jnp.float32)
        + b_ref[...], 0.0)
    h0_ref[...] = hv
    hp_ref[...] = hv * di
    dinv_ref[...] = di

  grid = (n // bm,)
  return pl.pallas_call(
      body,
      grid=grid,
      in_specs=[
          pl.BlockSpec((bm, d), lambda i: (i, 0)),
          pl.BlockSpec((d, h), lambda i: (0, 0)),
          pl.BlockSpec((1, h), lambda i: (0, 0)),
          pl.BlockSpec((NC, bm, 128), lambda i: (0, i, 0)),
      ],
      out_specs=[
          pl.BlockSpec((bm, h), lambda i: (i, 0)),
          pl.BlockSpec((bm, h), lambda i: (i, 0)),
          pl.BlockSpec((bm, 1), lambda i: (i, 0)),
      ],
      out_shape=[
          jax.ShapeDtypeStruct((n, h), jnp.float32),
          jax.ShapeDtypeStruct((n, h), jnp.float32),
          jax.ShapeDtypeStruct((n, 1), jnp.float32),
      ],
  )(x, w1, b1, cnt)


def _tc_layer(s2, hp, x0, dinv, w, beta, n, h, bm):
  def body(s_ref, hp_ref, x0_ref, di_ref, w_ref, h_ref, hpn_ref):
    di = di_ref[...]
    agg = di * (s_ref[0] + s_ref[1] + hp_ref[...])
    hh = (1.0 - ALPHA) * agg + ALPHA * x0_ref[...]
    out = (1.0 - beta) * hh + beta * jnp.dot(
        hh, w_ref[...], preferred_element_type=jnp.float32)
    hnew = jnp.maximum(out, 0.0)
    h_ref[...] = hnew
    hpn_ref[...] = hnew * di

  grid = (n // bm,)
  return pl.pallas_call(
      body,
      grid=grid,
      in_specs=[
          pl.BlockSpec((NC, bm, h), lambda i: (0, i, 0)),
          pl.BlockSpec((bm, h), lambda i: (i, 0)),
          pl.BlockSpec((bm, h), lambda i: (i, 0)),
          pl.BlockSpec((bm, 1), lambda i: (i, 0)),
          pl.BlockSpec((h, h), lambda i: (0, 0)),
      ],
      out_specs=[
          pl.BlockSpec((bm, h), lambda i: (i, 0)),
          pl.BlockSpec((bm, h), lambda i: (i, 0)),
      ],
      out_shape=[
          jax.ShapeDtypeStruct((n, h), jnp.float32),
          jax.ShapeDtypeStruct((n, h), jnp.float32),
      ],
  )(s2, hp, x0, dinv, w)


def _tc_pool(hfin, batch2, w2, b2, n, h, c, g):
  def body(h_ref, b_ref, w_ref, bias_ref, out_ref):
    seg = lax.broadcasted_iota(jnp.int32, (g, n), 0)
    oh = (seg == b_ref[...]).astype(jnp.float32)
    sums = jnp.dot(oh, h_ref[...], preferred_element_type=jnp.float32)
    counts = jnp.sum(oh, axis=1, keepdims=True)
    pooled = sums / jnp.maximum(counts, 1.0)
    logits = jnp.dot(pooled, w_ref[...],
                     preferred_element_type=jnp.float32) + bias_ref[...]
    m = jnp.max(logits, axis=-1, keepdims=True)
    z = logits - m
    lse = jnp.log(jnp.sum(jnp.exp(z), axis=-1, keepdims=True))
    out_ref[...] = z - lse

  return pl.pallas_call(
      body,
      out_shape=jax.ShapeDtypeStruct((g, c), jnp.float32),
  )(hfin, batch2, w2, b2)


@jax.jit
def kernel(x, lin1_w, lin1_b, conv_w, lin2_w, lin2_b, edge_index, batch):
  n, d = x.shape
  h = lin1_w.shape[1]
  nlayers = conv_w.shape[0]
  c = lin2_w.shape[1]
  g = 64
  e = edge_index.shape[1]
  bm = n // 10
  npad = 10240

  nchunk = -(-e // (NW * CH))
  epad = NW * nchunk * CH - e
  src_flat = jnp.concatenate(
      [edge_index[0].astype(jnp.int32),
       jnp.zeros((epad,), jnp.int32)])
  dst_flat = jnp.concatenate(
      [edge_index[1].astype(jnp.int32),
       jnp.full((epad,), npad - 1, jnp.int32)])
  src_rs = src_flat.reshape(NW, nchunk, CH)
  dst_rs = dst_flat.reshape(NW, nchunk, CH)
  onesd = jnp.ones((CH, h), jnp.float32)
  zerosd = jnp.zeros((npad // NS, h), jnp.float32)

  cnt = _sc_count(dst_rs, onesd, zerosd, npad, h, nchunk)
  h0, hp, dinv = _tc_prep(x, lin1_w, lin1_b.reshape(1, h), cnt, n, d, h, bm)

  hcur, hpcur = h0, hp
  for l in range(nlayers):
    beta = float(math.log(THETA / (l + 1) + 1.0))
    s2 = _sc_spmm(hpcur, src_rs, dst_rs, zerosd, npad, h, nchunk)
    hcur, hpcur = _tc_layer(s2, hpcur, h0, dinv, conv_w[l], beta, n, h, bm)

  return _tc_pool(hcur, batch.reshape(1, n).astype(jnp.int32), lin2_w,
                  lin2_b.reshape(1, c), n, h, c, g)


# speedup vs baseline: 2.0127x; 1.1442x over previous
import functools
import math

import jax
import jax.numpy as jnp
from jax import lax
from jax.experimental import pallas as pl
from jax.experimental.pallas import tpu as pltpu
from jax.experimental.pallas import tpu_sc as plsc

ALPHA = 0.1
THETA = 0.5

NC = 2
NS = 16
NW = NC * NS

CH = 80


def _sc_count(dst_rs, ones, zeros, npad, d, nchunk):
  rp = npad // NS

  mesh = plsc.VectorSubcoreMesh(
      core_axis_name="c", subcore_axis_name="s", num_cores=NC, num_subcores=NS)

  @functools.partial(
      pl.kernel,
      mesh=mesh,
      out_type=jax.ShapeDtypeStruct((NC, npad, d), jnp.float32),
      scratch_types=[
          pltpu.VMEM((nchunk, CH), jnp.int32),
          pltpu.VMEM((CH, d), jnp.float32),
          pltpu.VMEM_SHARED((npad, d), jnp.float32),
          pltpu.SemaphoreType.DMA,
      ],
  )
  def k(dst_hbm, ones_hbm, zeros_hbm, out_hbm, dst_v, ones_v, cnt_sh, sem):
    cid = lax.axis_index("c")
    sid = lax.axis_index("s")
    wid = sid * NC + cid
    pltpu.sync_copy(zeros_hbm, cnt_sh.at[pl.ds(sid * rp, rp)])
    pltpu.sync_copy(dst_hbm.at[wid], dst_v)
    pltpu.sync_copy(ones_hbm, ones_v)
    plsc.subcore_barrier()

    def chunk(j, carry):
      pltpu.sync_copy(ones_v, cnt_sh.at[dst_v.at[j]], add=True)
      return carry

    lax.fori_loop(0, nchunk, chunk, 0)
    plsc.subcore_barrier()
    pltpu.sync_copy(cnt_sh.at[pl.ds(sid * rp, rp)],
                    out_hbm.at[cid, pl.ds(sid * rp, rp)])

  return k(dst_rs, ones, zeros)


def _sc_spmm(hp, src_rs, dst_rs, zeros, npad, d, nchunk):
  rp = npad // NS

  mesh = plsc.VectorSubcoreMesh(
      core_axis_name="c", subcore_axis_name="s", num_cores=NC, num_subcores=NS)

  @functools.partial(
      pl.kernel,
      mesh=mesh,
      out_type=jax.ShapeDtypeStruct((NC, npad, d), jnp.float32),
      scratch_types=[
          pltpu.VMEM((CH,), jnp.int32),
          pltpu.VMEM((CH,), jnp.int32),
          pltpu.VMEM((CH,), jnp.int32),
          pltpu.VMEM((CH,), jnp.int32),
          pltpu.VMEM((CH,), jnp.int32),
          pltpu.VMEM((CH, d), jnp.float32),
          pltpu.VMEM((CH, d), jnp.float32),
          pltpu.VMEM_SHARED((npad, d), jnp.float32),
          pltpu.SemaphoreType.DMA,
          pltpu.SemaphoreType.DMA,
          pltpu.SemaphoreType.DMA,
          pltpu.SemaphoreType.DMA,
          pltpu.SemaphoreType.DMA,
          pltpu.SemaphoreType.DMA,
      ],
  )
  def k(hp_hbm, src_hbm, dst_hbm, zeros_hbm, out_hbm,
        si0, si1, di0, di1, di2, rows0_v, rows1_v, agg_sh,
        is0, is1, gs0, gs1, ss0, ss1):
    cid = lax.axis_index("c")
    sid = lax.axis_index("s")
    wid = sid * NC + cid
    si = (si0, si1)
    di = (di0, di1, di2)
    rows = (rows0_v, rows1_v)
    isem = (is0, is1)
    gsem = (gs0, gs1)
    ssem = (ss0, ss1)
    pltpu.sync_copy(zeros_hbm, agg_sh.at[pl.ds(sid * rp, rp)])
    plsc.subcore_barrier()

    def stage(j, k):
      b2, b3 = k % 2, k % 3
      o2, n3 = 1 - b2, (k + 1) % 3
      pltpu.make_async_copy(src_hbm.at[wid, j], si[b2], isem[b2]).wait()
      pltpu.make_async_copy(dst_hbm.at[wid, j], di[b3], isem[b2]).wait()

      @pl.when(j > 1)
      def _():
        pltpu.make_async_copy(rows[b2], agg_sh.at[di[b3]], ssem[b2]).wait()

      pltpu.async_copy(hp_hbm.at[si[b2]], rows[b2], gsem[b2])

      @pl.when(j > 0)
      def _():
        pltpu.make_async_copy(hp_hbm.at[si[o2]], rows[o2], gsem[o2]).wait()
        pltpu.async_copy(rows[o2], agg_sh.at[di[(k - 1) % 3]], ssem[o2],
                         add=True)

      @pl.when(j + 1 < nchunk)
      def _():
        pltpu.async_copy(src_hbm.at[wid, j + 1], si[o2], isem[o2])
        pltpu.async_copy(dst_hbm.at[wid, j + 1], di[n3], isem[o2])

    pltpu.async_copy(src_hbm.at[wid, 0], si0, is0)
    pltpu.async_copy(dst_hbm.at[wid, 0], di0, is0)

    def step(j, carry):
      m = lax.rem(j, 6)
      lax.switch(m, [lambda k=k: stage(j, k) for k in range(6)])
      return carry

    lax.fori_loop(0, nchunk, step, 0)

    last = nchunk - 1
    b2 = last % 2
    pltpu.make_async_copy(hp_hbm.at[si[b2]], rows[b2], gsem[b2]).wait()
    pltpu.async_copy(rows[b2], agg_sh.at[di[last % 3]], ssem[b2], add=True)
    pltpu.make_async_copy(rows[1 - b2], agg_sh.at[di[0]], ssem[1 - b2]).wait()
    pltpu.make_async_copy(rows[b2], agg_sh.at[di[0]], ssem[b2]).wait()
    plsc.subcore_barrier()
    pltpu.sync_copy(agg_sh.at[pl.ds(sid * rp, rp)],
                    out_hbm.at[cid, pl.ds(sid * rp, rp)])

  return k(hp, src_rs, dst_rs, zeros)


def _tc_prep(x, w1, b1, cnt, n, d, h, bm):
  def body(x_ref, w_ref, b_ref, cnt_ref, h0_ref, hp_ref, dinv_ref):
    deg = cnt_ref[0, :, 0:1] + cnt_ref[1, :, 0:1] + 1.0
    di = lax.rsqrt(deg)
    hv = jnp.maximum(
        jnp.dot(x_ref[...], w_ref[...], preferred_element_type=jnp.float32)
        + b_ref[...], 0.0)
    h0_ref[...] = hv
    hp_ref[...] = hv * di
    dinv_ref[...] = di

  grid = (n // bm,)
  return pl.pallas_call(
      body,
      grid=grid,
      in_specs=[
          pl.BlockSpec((bm, d), lambda i: (i, 0)),
          pl.BlockSpec((d, h), lambda i: (0, 0)),
          pl.BlockSpec((1, h), lambda i: (0, 0)),
          pl.BlockSpec((NC, bm, 128), lambda i: (0, i, 0)),
      ],
      out_specs=[
          pl.BlockSpec((bm, h), lambda i: (i, 0)),
          pl.BlockSpec((bm, h), lambda i: (i, 0)),
          pl.BlockSpec((bm, 1), lambda i: (i, 0)),
      ],
      out_shape=[
          jax.ShapeDtypeStruct((n, h), jnp.float32),
          jax.ShapeDtypeStruct((n, h), jnp.float32),
          jax.ShapeDtypeStruct((n, 1), jnp.float32),
      ],
  )(x, w1, b1, cnt)


def _tc_layer(s2, hp, x0, dinv, w, beta, n, h, bm):
  def body(s_ref, hp_ref, x0_ref, di_ref, w_ref, h_ref, hpn_ref):
    di = di_ref[...]
    agg = di * (s_ref[0] + s_ref[1] + hp_ref[...])
    hh = (1.0 - ALPHA) * agg + ALPHA * x0_ref[...]
    out = (1.0 - beta) * hh + beta * jnp.dot(
        hh, w_ref[...], preferred_element_type=jnp.float32)
    hnew = jnp.maximum(out, 0.0)
    h_ref[...] = hnew
    hpn_ref[...] = hnew * di

  grid = (n // bm,)
  return pl.pallas_call(
      body,
      grid=grid,
      in_specs=[
          pl.BlockSpec((NC, bm, h), lambda i: (0, i, 0)),
          pl.BlockSpec((bm, h), lambda i: (i, 0)),
          pl.BlockSpec((bm, h), lambda i: (i, 0)),
          pl.BlockSpec((bm, 1), lambda i: (i, 0)),
          pl.BlockSpec((h, h), lambda i: (0, 0)),
      ],
      out_specs=[
          pl.BlockSpec((bm, h), lambda i: (i, 0)),
          pl.BlockSpec((bm, h), lambda i: (i, 0)),
      ],
      out_shape=[
          jax.ShapeDtypeStruct((n, h), jnp.float32),
          jax.ShapeDtypeStruct((n, h), jnp.float32),
      ],
  )(s2, hp, x0, dinv, w)


def _tc_pool(hfin, batch2, w2, b2, n, h, c, g):
  def body(h_ref, b_ref, w_ref, bias_ref, out_ref):
    seg = lax.broadcasted_iota(jnp.int32, (g, n), 0)
    oh = (seg == b_ref[...]).astype(jnp.float32)
    sums = jnp.dot(oh, h_ref[...], preferred_element_type=jnp.float32)
    counts = jnp.sum(oh, axis=1, keepdims=True)
    pooled = sums / jnp.maximum(counts, 1.0)
    logits = jnp.dot(pooled, w_ref[...],
                     preferred_element_type=jnp.float32) + bias_ref[...]
    m = jnp.max(logits, axis=-1, keepdims=True)
    z = logits - m
    lse = jnp.log(jnp.sum(jnp.exp(z), axis=-1, keepdims=True))
    out_ref[...] = z - lse

  return pl.pallas_call(
      body,
      out_shape=jax.ShapeDtypeStruct((g, c), jnp.float32),
  )(hfin, batch2, w2, b2)


@jax.jit
def kernel(x, lin1_w, lin1_b, conv_w, lin2_w, lin2_b, edge_index, batch):
  n, d = x.shape
  h = lin1_w.shape[1]
  nlayers = conv_w.shape[0]
  c = lin2_w.shape[1]
  g = 64
  e = edge_index.shape[1]
  bm = n // 10
  npad = 10240

  nchunk = -(-e // (NW * CH))
  epad = NW * nchunk * CH - e
  src_flat = jnp.concatenate(
      [edge_index[0].astype(jnp.int32),
       jnp.zeros((epad,), jnp.int32)])
  dst_flat = jnp.concatenate(
      [edge_index[1].astype(jnp.int32),
       jnp.full((epad,), npad - 1, jnp.int32)])
  src_rs = src_flat.reshape(NW, nchunk, CH)
  dst_rs = dst_flat.reshape(NW, nchunk, CH)
  onesd = jnp.ones((CH, h), jnp.float32)
  zerosd = jnp.zeros((npad // NS, h), jnp.float32)

  cnt = _sc_count(dst_rs, onesd, zerosd, npad, h, nchunk)
  h0, hp, dinv = _tc_prep(x, lin1_w, lin1_b.reshape(1, h), cnt, n, d, h, bm)

  hcur, hpcur = h0, hp
  for l in range(nlayers):
    beta = float(math.log(THETA / (l + 1) + 1.0))
    s2 = _sc_spmm(hpcur, src_rs, dst_rs, zerosd, npad, h, nchunk)
    hcur, hpcur = _tc_layer(s2, hpcur, h0, dinv, conv_w[l], beta, n, h, bm)

  return _tc_pool(hcur, batch.reshape(1, n).astype(jnp.int32), lin2_w,
                  lin2_b.reshape(1, c), n, h, c, g)


# speedup vs baseline: 2.0154x; 1.0014x over previous
import functools
import math

import jax
import jax.numpy as jnp
from jax import lax
from jax.experimental import pallas as pl
from jax.experimental.pallas import tpu as pltpu
from jax.experimental.pallas import tpu_sc as plsc

ALPHA = 0.1
THETA = 0.5

NC = 2
NS = 16
NW = NC * NS

CH = 80


def _sc_count(dst_rs, ones, zeros, npad, d, nchunk):
  rp = npad // NS

  mesh = plsc.VectorSubcoreMesh(
      core_axis_name="c", subcore_axis_name="s", num_cores=NC, num_subcores=NS)

  @functools.partial(
      pl.kernel,
      mesh=mesh,
      out_type=jax.ShapeDtypeStruct((NC, npad, d), jnp.float32),
      scratch_types=[
          pltpu.VMEM((nchunk, CH), jnp.int32),
          pltpu.VMEM((CH, d), jnp.float32),
          pltpu.VMEM_SHARED((npad, d), jnp.float32),
          pltpu.SemaphoreType.DMA,
      ],
  )
  def k(dst_hbm, ones_hbm, zeros_hbm, out_hbm, dst_v, ones_v, cnt_sh, sem):
    cid = lax.axis_index("c")
    sid = lax.axis_index("s")
    wid = sid * NC + cid
    pltpu.sync_copy(zeros_hbm, cnt_sh.at[pl.ds(sid * rp, rp)])
    pltpu.sync_copy(dst_hbm.at[wid], dst_v)
    pltpu.sync_copy(ones_hbm, ones_v)
    plsc.subcore_barrier()

    def chunk(j, carry):
      pltpu.async_copy(ones_v, cnt_sh.at[dst_v.at[j]], sem, add=True)
      return carry

    lax.fori_loop(0, nchunk, chunk, 0)

    def drain(j, carry):
      pltpu.make_async_copy(ones_v, cnt_sh.at[dst_v.at[0]], sem).wait()
      return carry

    lax.fori_loop(0, nchunk, drain, 0)
    plsc.subcore_barrier()
    pltpu.sync_copy(cnt_sh.at[pl.ds(sid * rp, rp)],
                    out_hbm.at[cid, pl.ds(sid * rp, rp)])

  return k(dst_rs, ones, zeros)


def _sc_spmm(hp, src_rs, dst_rs, zeros, npad, d, nchunk):
  rp = npad // NS

  mesh = plsc.VectorSubcoreMesh(
      core_axis_name="c", subcore_axis_name="s", num_cores=NC, num_subcores=NS)

  @functools.partial(
      pl.kernel,
      mesh=mesh,
      out_type=jax.ShapeDtypeStruct((NC, npad, d), jnp.float32),
      scratch_types=[
          pltpu.VMEM((CH,), jnp.int32),
          pltpu.VMEM((CH,), jnp.int32),
          pltpu.VMEM((CH,), jnp.int32),
          pltpu.VMEM((CH,), jnp.int32),
          pltpu.VMEM((CH,), jnp.int32),
          pltpu.VMEM((CH, d), jnp.float32),
          pltpu.VMEM((CH, d), jnp.float32),
          pltpu.VMEM_SHARED((npad, d), jnp.float32),
          pltpu.SemaphoreType.DMA,
          pltpu.SemaphoreType.DMA,
          pltpu.SemaphoreType.DMA,
          pltpu.SemaphoreType.DMA,
          pltpu.SemaphoreType.DMA,
          pltpu.SemaphoreType.DMA,
      ],
  )
  def k(hp_hbm, src_hbm, dst_hbm, zeros_hbm, out_hbm,
        si0, si1, di0, di1, di2, rows0_v, rows1_v, agg_sh,
        is0, is1, gs0, gs1, ss0, ss1):
    cid = lax.axis_index("c")
    sid = lax.axis_index("s")
    wid = sid * NC + cid
    si = (si0, si1)
    di = (di0, di1, di2)
    rows = (rows0_v, rows1_v)
    isem = (is0, is1)
    gsem = (gs0, gs1)
    ssem = (ss0, ss1)
    pltpu.sync_copy(zeros_hbm, agg_sh.at[pl.ds(sid * rp, rp)])
    plsc.subcore_barrier()

    def stage(j, k):
      b2, b3 = k % 2, k % 3
      o2, n3 = 1 - b2, (k + 1) % 3
      pltpu.make_async_copy(src_hbm.at[wid, j], si[b2], isem[b2]).wait()
      pltpu.make_async_copy(dst_hbm.at[wid, j], di[b3], isem[b2]).wait()

      @pl.when(j > 1)
      def _():
        pltpu.make_async_copy(rows[b2], agg_sh.at[di[b3]], ssem[b2]).wait()

      pltpu.async_copy(hp_hbm.at[si[b2]], rows[b2], gsem[b2])

      @pl.when(j > 0)
      def _():
        pltpu.make_async_copy(hp_hbm.at[si[o2]], rows[o2], gsem[o2]).wait()
        pltpu.async_copy(rows[o2], agg_sh.at[di[(k - 1) % 3]], ssem[o2],
                         add=True)

      @pl.when(j + 1 < nchunk)
      def _():
        pltpu.async_copy(src_hbm.at[wid, j + 1], si[o2], isem[o2])
        pltpu.async_copy(dst_hbm.at[wid, j + 1], di[n3], isem[o2])

    pltpu.async_copy(src_hbm.at[wid, 0], si0, is0)
    pltpu.async_copy(dst_hbm.at[wid, 0], di0, is0)

    def step(j, carry):
      m = lax.rem(j, 6)
      lax.switch(m, [lambda k=k: stage(j, k) for k in range(6)])
      return carry

    lax.fori_loop(0, nchunk, step, 0)

    last = nchunk - 1
    b2 = last % 2
    pltpu.make_async_copy(hp_hbm.at[si[b2]], rows[b2], gsem[b2]).wait()
    pltpu.async_copy(rows[b2], agg_sh.at[di[last % 3]], ssem[b2], add=True)
    pltpu.make_async_copy(rows[1 - b2], agg_sh.at[di[0]], ssem[1 - b2]).wait()
    pltpu.make_async_copy(rows[b2], agg_sh.at[di[0]], ssem[b2]).wait()
    plsc.subcore_barrier()
    pltpu.sync_copy(agg_sh.at[pl.ds(sid * rp, rp)],
                    out_hbm.at[cid, pl.ds(sid * rp, rp)])

  return k(hp, src_rs, dst_rs, zeros)


def _tc_prep(x, w1, b1, cnt, n, d, h, bm):
  def body(x_ref, w_ref, b_ref, cnt_ref, h0_ref, hp_ref, dinv_ref):
    deg = cnt_ref[0, :, 0:1] + cnt_ref[1, :, 0:1] + 1.0
    di = lax.rsqrt(deg)
    hv = jnp.maximum(
        jnp.dot(x_ref[...], w_ref[...], preferred_element_type=jnp.float32)
        + b_ref[...], 0.0)
    h0_ref[...] = hv
    hp_ref[...] = hv * di
    dinv_ref[...] = di

  grid = (n // bm,)
  return pl.pallas_call(
      body,
      grid=grid,
      in_specs=[
          pl.BlockSpec((bm, d), lambda i: (i, 0)),
          pl.BlockSpec((d, h), lambda i: (0, 0)),
          pl.BlockSpec((1, h), lambda i: (0, 0)),
          pl.BlockSpec((NC, bm, 128), lambda i: (0, i, 0)),
      ],
      out_specs=[
          pl.BlockSpec((bm, h), lambda i: (i, 0)),
          pl.BlockSpec((bm, h), lambda i: (i, 0)),
          pl.BlockSpec((bm, 1), lambda i: (i, 0)),
      ],
      out_shape=[
          jax.ShapeDtypeStruct((n, h), jnp.float32),
          jax.ShapeDtypeStruct((n, h), jnp.float32),
          jax.ShapeDtypeStruct((n, 1), jnp.float32),
      ],
  )(x, w1, b1, cnt)


def _tc_layer(s2, hp, x0, dinv, w, beta, n, h, bm):
  def body(s_ref, hp_ref, x0_ref, di_ref, w_ref, h_ref, hpn_ref):
    di = di_ref[...]
    agg = di * (s_ref[0] + s_ref[1] + hp_ref[...])
    hh = (1.0 - ALPHA) * agg + ALPHA * x0_ref[...]
    out = (1.0 - beta) * hh + beta * jnp.dot(
        hh, w_ref[...], preferred_element_type=jnp.float32)
    hnew = jnp.maximum(out, 0.0)
    h_ref[...] = hnew
    hpn_ref[...] = hnew * di

  grid = (n // bm,)
  return pl.pallas_call(
      body,
      grid=grid,
      in_specs=[
          pl.BlockSpec((NC, bm, h), lambda i: (0, i, 0)),
          pl.BlockSpec((bm, h), lambda i: (i, 0)),
          pl.BlockSpec((bm, h), lambda i: (i, 0)),
          pl.BlockSpec((bm, 1), lambda i: (i, 0)),
          pl.BlockSpec((h, h), lambda i: (0, 0)),
      ],
      out_specs=[
          pl.BlockSpec((bm, h), lambda i: (i, 0)),
          pl.BlockSpec((bm, h), lambda i: (i, 0)),
      ],
      out_shape=[
          jax.ShapeDtypeStruct((n, h), jnp.float32),
          jax.ShapeDtypeStruct((n, h), jnp.float32),
      ],
  )(s2, hp, x0, dinv, w)


def _tc_pool(hfin, batch2, w2, b2, n, h, c, g):
  def body(h_ref, b_ref, w_ref, bias_ref, out_ref):
    seg = lax.broadcasted_iota(jnp.int32, (g, n), 0)
    oh = (seg == b_ref[...]).astype(jnp.float32)
    sums = jnp.dot(oh, h_ref[...], preferred_element_type=jnp.float32)
    counts = jnp.sum(oh, axis=1, keepdims=True)
    pooled = sums / jnp.maximum(counts, 1.0)
    logits = jnp.dot(pooled, w_ref[...],
                     preferred_element_type=jnp.float32) + bias_ref[...]
    m = jnp.max(logits, axis=-1, keepdims=True)
    z = logits - m
    lse = jnp.log(jnp.sum(jnp.exp(z), axis=-1, keepdims=True))
    out_ref[...] = z - lse

  return pl.pallas_call(
      body,
      out_shape=jax.ShapeDtypeStruct((g, c), jnp.float32),
  )(hfin, batch2, w2, b2)


@jax.jit
def kernel(x, lin1_w, lin1_b, conv_w, lin2_w, lin2_b, edge_index, batch):
  n, d = x.shape
  h = lin1_w.shape[1]
  nlayers = conv_w.shape[0]
  c = lin2_w.shape[1]
  g = 64
  e = edge_index.shape[1]
  bm = n // 10
  npad = 10240

  nchunk = -(-e // (NW * CH))
  epad = NW * nchunk * CH - e
  src_flat = jnp.concatenate(
      [edge_index[0].astype(jnp.int32),
       jnp.zeros((epad,), jnp.int32)])
  dst_flat = jnp.concatenate(
      [edge_index[1].astype(jnp.int32),
       jnp.full((epad,), npad - 1, jnp.int32)])
  src_rs = src_flat.reshape(NW, nchunk, CH)
  dst_rs = dst_flat.reshape(NW, nchunk, CH)
  onesd = jnp.ones((CH, h), jnp.float32)
  zerosd = jnp.zeros((npad // NS, h), jnp.float32)

  cnt = _sc_count(dst_rs, onesd, zerosd, npad, h, nchunk)
  h0, hp, dinv = _tc_prep(x, lin1_w, lin1_b.reshape(1, h), cnt, n, d, h, bm)

  hcur, hpcur = h0, hp
  for l in range(nlayers):
    beta = float(math.log(THETA / (l + 1) + 1.0))
    s2 = _sc_spmm(hpcur, src_rs, dst_rs, zerosd, npad, h, nchunk)
    hcur, hpcur = _tc_layer(s2, hpcur, h0, dinv, conv_w[l], beta, n, h, bm)

  return _tc_pool(hcur, batch.reshape(1, n).astype(jnp.int32), lin2_w,
                  lin2_b.reshape(1, c), n, h, c, g)


# speedup vs baseline: 2.0207x; 1.0026x over previous
import functools
import math

import jax
import jax.numpy as jnp
from jax import lax
from jax.experimental import pallas as pl
from jax.experimental.pallas import tpu as pltpu
from jax.experimental.pallas import tpu_sc as plsc

ALPHA = 0.1
THETA = 0.5

NC = 2
NS = 16
NW = NC * NS

CH = 80


def _sc_count(dst_rs, ones, zeros, npad, d, nchunk):
  rp = npad // NS

  mesh = plsc.VectorSubcoreMesh(
      core_axis_name="c", subcore_axis_name="s", num_cores=NC, num_subcores=NS)

  @functools.partial(
      pl.kernel,
      mesh=mesh,
      out_type=jax.ShapeDtypeStruct((NC, npad, d), jnp.float32),
      scratch_types=[
          pltpu.VMEM((nchunk, CH), jnp.int32),
          pltpu.VMEM((CH, d), jnp.float32),
          pltpu.VMEM_SHARED((npad, d), jnp.float32),
          pltpu.SemaphoreType.DMA,
      ],
  )
  def k(dst_hbm, ones_hbm, zeros_hbm, out_hbm, dst_v, ones_v, cnt_sh, sem):
    cid = lax.axis_index("c")
    sid = lax.axis_index("s")
    wid = sid * NC + cid
    pltpu.sync_copy(zeros_hbm, cnt_sh.at[pl.ds(sid * rp, rp)])
    pltpu.sync_copy(dst_hbm.at[wid], dst_v)
    pltpu.sync_copy(ones_hbm, ones_v)
    plsc.subcore_barrier()

    def chunk(j, carry):
      pltpu.async_copy(ones_v, cnt_sh.at[dst_v.at[j]], sem, add=True)
      return carry

    lax.fori_loop(0, nchunk, chunk, 0)

    def drain(j, carry):
      pltpu.make_async_copy(ones_v, cnt_sh.at[dst_v.at[0]], sem).wait()
      return carry

    lax.fori_loop(0, nchunk, drain, 0)
    plsc.subcore_barrier()
    pltpu.sync_copy(cnt_sh.at[pl.ds(sid * rp, rp)],
                    out_hbm.at[cid, pl.ds(sid * rp, rp)])

  return k(dst_rs, ones, zeros)


def _sc_spmm(hp, ed_rs, zeros, npad, d, nchunk):
  rp = npad // NS

  mesh = plsc.VectorSubcoreMesh(
      core_axis_name="c", subcore_axis_name="s", num_cores=NC, num_subcores=NS)

  @functools.partial(
      pl.kernel,
      mesh=mesh,
      out_type=jax.ShapeDtypeStruct((NC, npad, d), jnp.float32),
      scratch_types=[
          pltpu.VMEM((2, CH), jnp.int32),
          pltpu.VMEM((2, CH), jnp.int32),
          pltpu.VMEM((2, CH), jnp.int32),
          pltpu.VMEM((CH, d), jnp.float32),
          pltpu.VMEM((CH, d), jnp.float32),
          pltpu.VMEM_SHARED((npad, d), jnp.float32),
          pltpu.SemaphoreType.DMA,
          pltpu.SemaphoreType.DMA,
          pltpu.SemaphoreType.DMA,
          pltpu.SemaphoreType.DMA,
          pltpu.SemaphoreType.DMA,
          pltpu.SemaphoreType.DMA,
          pltpu.SemaphoreType.DMA,
      ],
  )
  def k(hp_hbm, ed_hbm, zeros_hbm, out_hbm,
        ib0, ib1, ib2, rows0_v, rows1_v, agg_sh,
        is0, is1, is2, gs0, gs1, ss0, ss1):
    cid = lax.axis_index("c")
    sid = lax.axis_index("s")
    wid = sid * NC + cid
    ib = (ib0, ib1, ib2)
    rows = (rows0_v, rows1_v)
    isem = (is0, is1, is2)
    gsem = (gs0, gs1)
    ssem = (ss0, ss1)
    pltpu.sync_copy(zeros_hbm, agg_sh.at[pl.ds(sid * rp, rp)])
    plsc.subcore_barrier()

    def stage(j, k):
      b2, b3 = k % 2, k % 3
      o2, n3 = 1 - b2, (k + 1) % 3
      pltpu.make_async_copy(ed_hbm.at[wid, j], ib[b3], isem[b3]).wait()

      @pl.when(j > 1)
      def _():
        pltpu.make_async_copy(rows[b2], agg_sh.at[ib[b3].at[1]],
                              ssem[b2]).wait()

      pltpu.async_copy(hp_hbm.at[ib[b3].at[0]], rows[b2], gsem[b2])

      @pl.when(j > 0)
      def _():
        ob = ib[(k - 1) % 3]
        pltpu.make_async_copy(hp_hbm.at[ob.at[0]], rows[o2], gsem[o2]).wait()
        pltpu.async_copy(rows[o2], agg_sh.at[ob.at[1]], ssem[o2], add=True)

      @pl.when(j + 1 < nchunk)
      def _():
        pltpu.async_copy(ed_hbm.at[wid, j + 1], ib[n3], isem[n3])

    pltpu.async_copy(ed_hbm.at[wid, 0], ib0, is0)

    def step(j, carry):
      m = lax.rem(j, 6)
      lax.switch(m, [lambda k=k: stage(j, k) for k in range(6)])
      return carry

    lax.fori_loop(0, nchunk, step, 0)

    last = nchunk - 1
    b2 = last % 2
    lb = ib[last % 3]
    pltpu.make_async_copy(hp_hbm.at[lb.at[0]], rows[b2], gsem[b2]).wait()
    pltpu.async_copy(rows[b2], agg_sh.at[lb.at[1]], ssem[b2], add=True)
    pltpu.make_async_copy(rows[1 - b2], agg_sh.at[lb.at[1]],
                          ssem[1 - b2]).wait()
    pltpu.make_async_copy(rows[b2], agg_sh.at[lb.at[1]], ssem[b2]).wait()
    plsc.subcore_barrier()
    pltpu.sync_copy(agg_sh.at[pl.ds(sid * rp, rp)],
                    out_hbm.at[cid, pl.ds(sid * rp, rp)])

  return k(hp, ed_rs, zeros)


def _tc_prep(x, w1, b1, cnt, n, d, h, bm):
  def body(x_ref, w_ref, b_ref, cnt_ref, h0_ref, hp_ref, dinv_ref):
    deg = cnt_ref[0, :, 0:1] + cnt_ref[1, :, 0:1] + 1.0
    di = lax.rsqrt(deg)
    hv = jnp.maximum(
        jnp.dot(x_ref[...], w_ref[...], preferred_element_type=jnp.float32)
        + b_ref[...], 0.0)
    h0_ref[...] = hv
    hp_ref[...] = hv * di
    dinv_ref[...] = di

  grid = (n // bm,)
  return pl.pallas_call(
      body,
      grid=grid,
      in_specs=[
          pl.BlockSpec((bm, d), lambda i: (i, 0)),
          pl.BlockSpec((d, h), lambda i: (0, 0)),
          pl.BlockSpec((1, h), lambda i: (0, 0)),
          pl.BlockSpec((NC, bm, 128), lambda i: (0, i, 0)),
      ],
      out_specs=[
          pl.BlockSpec((bm, h), lambda i: (i, 0)),
          pl.BlockSpec((bm, h), lambda i: (i, 0)),
          pl.BlockSpec((bm, 1), lambda i: (i, 0)),
      ],
      out_shape=[
          jax.ShapeDtypeStruct((n, h), jnp.float32),
          jax.ShapeDtypeStruct((n, h), jnp.float32),
          jax.ShapeDtypeStruct((n, 1), jnp.float32),
      ],
  )(x, w1, b1, cnt)


def _tc_layer(s2, hp, x0, dinv, w, beta, n, h, bm):
  def body(s_ref, hp_ref, x0_ref, di_ref, w_ref, h_ref, hpn_ref):
    di = di_ref[...]
    agg = di * (s_ref[0] + s_ref[1] + hp_ref[...])
    hh = (1.0 - ALPHA) * agg + ALPHA * x0_ref[...]
    out = (1.0 - beta) * hh + beta * jnp.dot(
        hh, w_ref[...], preferred_element_type=jnp.float32)
    hnew = jnp.maximum(out, 0.0)
    h_ref[...] = hnew
    hpn_ref[...] = hnew * di

  grid = (n // bm,)
  return pl.pallas_call(
      body,
      grid=grid,
      in_specs=[
          pl.BlockSpec((NC, bm, h), lambda i: (0, i, 0)),
          pl.BlockSpec((bm, h), lambda i: (i, 0)),
          pl.BlockSpec((bm, h), lambda i: (i, 0)),
          pl.BlockSpec((bm, 1), lambda i: (i, 0)),
          pl.BlockSpec((h, h), lambda i: (0, 0)),
      ],
      out_specs=[
          pl.BlockSpec((bm, h), lambda i: (i, 0)),
          pl.BlockSpec((bm, h), lambda i: (i, 0)),
      ],
      out_shape=[
          jax.ShapeDtypeStruct((n, h), jnp.float32),
          jax.ShapeDtypeStruct((n, h), jnp.float32),
      ],
  )(s2, hp, x0, dinv, w)


def _tc_pool(hfin, batch2, w2, b2, n, h, c, g):
  def body(h_ref, b_ref, w_ref, bias_ref, out_ref):
    seg = lax.broadcasted_iota(jnp.int32, (g, n), 0)
    oh = (seg == b_ref[...]).astype(jnp.float32)
    sums = jnp.dot(oh, h_ref[...], preferred_element_type=jnp.float32)
    counts = jnp.sum(oh, axis=1, keepdims=True)
    pooled = sums / jnp.maximum(counts, 1.0)
    logits = jnp.dot(pooled, w_ref[...],
                     preferred_element_type=jnp.float32) + bias_ref[...]
    m = jnp.max(logits, axis=-1, keepdims=True)
    z = logits - m
    lse = jnp.log(jnp.sum(jnp.exp(z), axis=-1, keepdims=True))
    out_ref[...] = z - lse

  return pl.pallas_call(
      body,
      out_shape=jax.ShapeDtypeStruct((g, c), jnp.float32),
  )(hfin, batch2, w2, b2)


@jax.jit
def kernel(x, lin1_w, lin1_b, conv_w, lin2_w, lin2_b, edge_index, batch):
  n, d = x.shape
  h = lin1_w.shape[1]
  nlayers = conv_w.shape[0]
  c = lin2_w.shape[1]
  g = 64
  e = edge_index.shape[1]
  bm = n // 10
  npad = 10240

  nchunk = -(-e // (NW * CH))
  epad = NW * nchunk * CH - e
  src_flat = jnp.concatenate(
      [edge_index[0].astype(jnp.int32),
       jnp.zeros((epad,), jnp.int32)])
  dst_flat = jnp.concatenate(
      [edge_index[1].astype(jnp.int32),
       jnp.full((epad,), npad - 1, jnp.int32)])
  src_rs = src_flat.reshape(NW, nchunk, CH)
  dst_rs = dst_flat.reshape(NW, nchunk, CH)
  ed_rs = jnp.stack([src_rs, dst_rs], axis=2)
  onesd = jnp.ones((CH, h), jnp.float32)
  zerosd = jnp.zeros((npad // NS, h), jnp.float32)

  cnt = _sc_count(dst_rs, onesd, zerosd, npad, h, nchunk)
  h0, hp, dinv = _tc_prep(x, lin1_w, lin1_b.reshape(1, h), cnt, n, d, h, bm)

  hcur, hpcur = h0, hp
  for l in range(nlayers):
    beta = float(math.log(THETA / (l + 1) + 1.0))
    s2 = _sc_spmm(hpcur, ed_rs, zerosd, npad, h, nchunk)
    hcur, hpcur = _tc_layer(s2, hpcur, h0, dinv, conv_w[l], beta, n, h, bm)

  return _tc_pool(hcur, batch.reshape(1, n).astype(jnp.int32), lin2_w,
                  lin2_b.reshape(1, c), n, h, c, g)
